# baseline (speedup 1.0000x reference)
import jax
import jax.numpy as jnp
from jax import lax
from jax.experimental import pallas as pl
from jax.experimental.pallas import tpu as pltpu
from jax.experimental.pallas import tpu_sc as plsc

NUM_STATES = 1000000
NUM_ACTIONS = 64
BATCH = 16384

NC = 2
NS = 16
NW = NC * NS
LANES = 16
BPW = BATCH // NW
CHUNK = 128
NCHUNK = BPW // CHUNK
COLS = NUM_ACTIONS // LANES

_NEG = -1000000000.0


def _body(obs_hbm, masks_hbm, table_hbm, out_hbm, idx_v, rows_v, masks_v,
          gsem, msem):
    wid = lax.axis_index("s") * NC + lax.axis_index("c")
    base = wid * BPW

    pltpu.sync_copy(obs_hbm.at[pl.ds(base, BPW)], idx_v)

    mcopy = pltpu.async_copy(masks_hbm.at[pl.ds(base, BPW)], masks_v, msem)
    gathers = [
        pltpu.async_copy(
            table_hbm.at[idx_v.at[pl.ds(j * CHUNK, CHUNK)]],
            rows_v.at[pl.ds(j * CHUNK, CHUNK)],
            gsem,
        )
        for j in range(NCHUNK)
    ]
    for g in gathers:
        g.wait()
    mcopy.wait()

    def row(r, carry):
        for c in range(COLS):
            sl = pl.ds(c * LANES, LANES)
            q = rows_v[r, sl]
            m = masks_v[r, sl]
            rows_v[r, sl] = m * q + (1.0 - m) * _NEG
        return carry

    lax.fori_loop(0, BPW, row, 0, unroll=2)

    pltpu.sync_copy(rows_v, out_hbm.at[pl.ds(base, BPW)])


def kernel(observations, action_masks, q_table):
    obs = observations.reshape(-1).astype(jnp.int32)
    mesh = plsc.VectorSubcoreMesh(
        core_axis_name="c", subcore_axis_name="s", num_cores=NC,
        num_subcores=NS)
    run = pl.kernel(
        _body,
        out_type=jax.ShapeDtypeStruct((BATCH, NUM_ACTIONS), jnp.float32),
        mesh=mesh,
        scratch_types=[
            pltpu.VMEM((BPW,), jnp.int32),
            pltpu.VMEM((BPW, NUM_ACTIONS), jnp.float32),
            pltpu.VMEM((BPW, NUM_ACTIONS), jnp.float32),
            pltpu.SemaphoreType.DMA,
            pltpu.SemaphoreType.DMA,
        ],
        compiler_params=pltpu.CompilerParams(use_tc_tiling_on_sc=False),
    )
    return run(obs, action_masks, q_table)


# speedup vs baseline: 1.0052x; 1.0052x over previous
import jax
import jax.numpy as jnp
from jax import lax
from jax.experimental import pallas as pl
from jax.experimental.pallas import tpu as pltpu
from jax.experimental.pallas import tpu_sc as plsc

NUM_STATES = 1000000
NUM_ACTIONS = 64
BATCH = 16384

NC = 2
NS = 16
NW = NC * NS
LANES = 16
BPW = BATCH // NW
CHUNK = 128
NCHUNK = BPW // CHUNK
KB = CHUNK // LANES

_NEG = -1000000000.0


def _body(obs_hbm, masksT_hbm, tab_hbm, outT_hbm,
          idx_v, jdx_v, cb_v, rows_v, m_v, out_v, gsem0, gsem1, msem):
    wid = lax.axis_index("s") * NC + lax.axis_index("c")
    base = wid * BPW

    pltpu.sync_copy(obs_hbm.at[pl.ds(base, BPW)], idx_v)
    mcopy = pltpu.async_copy(masksT_hbm.at[:, pl.ds(base, BPW)], m_v, msem)

    for k in range(BPW // LANES):
        sl = pl.ds(k * LANES, LANES)
        s = idx_v[sl]
        jdx_v[sl] = lax.shift_right_logical(s, 1)
        cb_v[sl] = lax.shift_left(lax.bitwise_and(s, 1), 6)

    gsems = [gsem0, gsem1]

    def fire(i):
        return pltpu.async_copy(
            tab_hbm.at[jdx_v.at[pl.ds(i * CHUNK, CHUNK)]],
            rows_v.at[i % 2],
            gsems[i % 2],
        )

    iota16 = lax.iota(jnp.int32, 16)
    copies = [fire(0), None]
    mcopy.wait()

    for i in range(NCHUNK):
        if i + 1 < NCHUNK:
            copies[(i + 1) % 2] = fire(i + 1)
        copies[i % 2].wait()
        buf = rows_v.at[i % 2]

        def kblock(k, carry, i=i, buf=buf):
            b0 = i * CHUNK + k * LANES
            rowi = iota16 + k * LANES
            h16 = cb_v[pl.ds(b0, LANES)]

            def abody(a, c2):
                q = plsc.load_gather(buf, [rowi, h16 + a])
                m = m_v[a, pl.ds(b0, LANES)]
                out_v[a, pl.ds(b0, LANES)] = m * q + (1.0 - m) * _NEG
                return c2

            lax.fori_loop(0, NUM_ACTIONS, abody, 0)
            return carry

        lax.fori_loop(0, KB, kblock, 0)

    pltpu.sync_copy(out_v, outT_hbm.at[:, pl.ds(base, BPW)])


def kernel(observations, action_masks, q_table):
    obs = observations.reshape(-1).astype(jnp.int32)
    tab = q_table.reshape(NUM_STATES // 2, 2 * NUM_ACTIONS)
    masksT = action_masks.T
    mesh = plsc.VectorSubcoreMesh(
        core_axis_name="c", subcore_axis_name="s", num_cores=NC,
        num_subcores=NS)
    run = pl.kernel(
        _body,
        out_type=jax.ShapeDtypeStruct((NUM_ACTIONS, BATCH), jnp.float32),
        mesh=mesh,
        scratch_types=[
            pltpu.VMEM((BPW,), jnp.int32),
            pltpu.VMEM((BPW,), jnp.int32),
            pltpu.VMEM((BPW,), jnp.int32),
            pltpu.VMEM((2, CHUNK, 2 * NUM_ACTIONS), jnp.float32),
            pltpu.VMEM((NUM_ACTIONS, BPW), jnp.float32),
            pltpu.VMEM((NUM_ACTIONS, BPW), jnp.float32),
            pltpu.SemaphoreType.DMA,
            pltpu.SemaphoreType.DMA,
            pltpu.SemaphoreType.DMA,
        ],
        compiler_params=pltpu.CompilerParams(needs_layout_passes=False),
    )
    outT = run(obs, masksT, tab)
    return outT.T


# speedup vs baseline: 2.8919x; 2.8770x over previous
import jax
import jax.numpy as jnp
from jax import lax
from jax.experimental import pallas as pl
from jax.experimental.pallas import tpu as pltpu
from jax.experimental.pallas import tpu_sc as plsc

NUM_STATES = 1000000
NUM_ACTIONS = 64
BATCH = 16384

NC = 2
NS = 16
NW = NC * NS
LANES = 16
BPW = BATCH // NW
NRING = 6

_NEG = -1000000000.0


def _body(obs_hbm, masksT_hbm, qT_hbm, outT_hbm,
          idx_v, slab_v, m_v, out_v, gsem, msem):
    wid = lax.axis_index("s") * NC + lax.axis_index("c")
    base = wid * BPW

    pltpu.sync_copy(obs_hbm.at[pl.ds(base, BPW)], idx_v)
    mcopy = pltpu.async_copy(masksT_hbm.at[:, pl.ds(base, BPW)], m_v, msem)

    iota16 = lax.iota(jnp.int32, LANES)

    def state_of(p):
        g = lax.div(p, LANES)
        j = lax.rem(p, LANES)
        s16 = idx_v[pl.ds(g * LANES, LANES)]
        return jnp.sum(jnp.where(iota16 == j, s16, 0))

    def fire(p):
        s = state_of(p)
        st = pl.multiple_of(lax.shift_left(lax.shift_right_logical(s, 7), 7), 128)
        slot = lax.rem(p, NRING)
        return pltpu.async_copy(
            qT_hbm.at[:, pl.ds(st, 128)], slab_v.at[slot], gsem.at[slot])

    for p in range(NRING - 1):
        fire(p)
    mcopy.wait()

    def step(p, carry):
        @pl.when(p + NRING - 1 < BPW)
        def _():
            fire(p + NRING - 1)

        s = state_of(p)
        slot = lax.rem(p, NRING)
        pltpu.make_async_copy(
            qT_hbm.at[:, pl.ds(0, 128)], slab_v.at[slot], gsem.at[slot]
        ).wait()
        lane = lax.bitwise_and(s, 127)
        lane_vec = iota16 * 0 + lane
        p_vec = iota16 * 0 + p
        buf = slab_v.at[slot]
        for g in range(NUM_ACTIONS // LANES):
            a_vec = iota16 + g * LANES
            q = plsc.load_gather(buf, [a_vec, lane_vec])
            m = plsc.load_gather(m_v, [a_vec, p_vec])
            res = m * q + (1.0 - m) * _NEG
            plsc.store_scatter(out_v, [a_vec, p_vec], res)
        return carry

    lax.fori_loop(0, BPW, step, 0)

    pltpu.sync_copy(out_v, outT_hbm.at[:, pl.ds(base, BPW)])


def kernel(observations, action_masks, q_table):
    obs = observations.reshape(-1).astype(jnp.int32)
    qT = q_table.T
    masksT = action_masks.T
    mesh = plsc.VectorSubcoreMesh(
        core_axis_name="c", subcore_axis_name="s", num_cores=NC,
        num_subcores=NS)
    run = pl.kernel(
        _body,
        out_type=jax.ShapeDtypeStruct((NUM_ACTIONS, BATCH), jnp.float32),
        mesh=mesh,
        scratch_types=[
            pltpu.VMEM((BPW,), jnp.int32),
            pltpu.VMEM((NRING, NUM_ACTIONS, 128), jnp.float32),
            pltpu.VMEM((NUM_ACTIONS, BPW), jnp.float32),
            pltpu.VMEM((NUM_ACTIONS, BPW), jnp.float32),
            pltpu.SemaphoreType.DMA((NRING,)),
            pltpu.SemaphoreType.DMA,
        ],
        compiler_params=pltpu.CompilerParams(needs_layout_passes=False),
    )
    outT = run(obs, masksT, qT)
    return outT.T


# speedup vs baseline: 3.1233x; 1.0800x over previous
import jax
import jax.numpy as jnp
from jax import lax
from jax.experimental import pallas as pl
from jax.experimental.pallas import tpu as pltpu
from jax.experimental.pallas import tpu_sc as plsc

NUM_STATES = 1000000
NUM_ACTIONS = 64
BATCH = 16384

NC = 2
NS = 16
NW = NC * NS
LANES = 16
BPW = BATCH // NW
NRING = 10

_NEG = -1000000000.0


def _body(obs_hbm, masksT_hbm, qT_hbm, outT_hbm,
          idx_v, slab_v, m_v, gsem, msem):
    out_v = m_v
    wid = lax.axis_index("s") * NC + lax.axis_index("c")
    base = wid * BPW

    pltpu.sync_copy(obs_hbm.at[pl.ds(base, BPW)], idx_v)
    mcopy = pltpu.async_copy(masksT_hbm.at[:, pl.ds(base, BPW)], m_v, msem)

    iota16 = lax.iota(jnp.int32, LANES)

    def state_of(p):
        g = lax.div(p, LANES)
        j = lax.rem(p, LANES)
        s16 = idx_v[pl.ds(g * LANES, LANES)]
        return jnp.sum(jnp.where(iota16 == j, s16, 0))

    def fire(p):
        s = state_of(p)
        st = pl.multiple_of(lax.shift_left(lax.shift_right_logical(s, 7), 7), 128)
        slot = lax.rem(p, NRING)
        return pltpu.async_copy(
            qT_hbm.at[:, pl.ds(st, 128)], slab_v.at[slot], gsem.at[slot])

    for p in range(NRING - 1):
        fire(p)
    mcopy.wait()

    def step(p, carry):
        @pl.when(p + NRING - 1 < BPW)
        def _():
            fire(p + NRING - 1)

        s = state_of(p)
        slot = lax.rem(p, NRING)
        pltpu.make_async_copy(
            qT_hbm.at[:, pl.ds(0, 128)], slab_v.at[slot], gsem.at[slot]
        ).wait()
        lane = lax.bitwise_and(s, 127)
        lane_vec = iota16 * 0 + lane
        p_vec = iota16 * 0 + p
        buf = slab_v.at[slot]
        for g in range(NUM_ACTIONS // LANES):
            a_vec = iota16 + g * LANES
            q = plsc.load_gather(buf, [a_vec, lane_vec])
            m = plsc.load_gather(m_v, [a_vec, p_vec])
            res = m * q + (1.0 - m) * _NEG
            plsc.store_scatter(out_v, [a_vec, p_vec], res)
        return carry

    lax.fori_loop(0, BPW, step, 0)

    pltpu.sync_copy(out_v, outT_hbm.at[:, pl.ds(base, BPW)])


def kernel(observations, action_masks, q_table):
    obs = observations.reshape(-1).astype(jnp.int32)
    qT = q_table.T
    masksT = action_masks.T
    mesh = plsc.VectorSubcoreMesh(
        core_axis_name="c", subcore_axis_name="s", num_cores=NC,
        num_subcores=NS)
    run = pl.kernel(
        _body,
        out_type=jax.ShapeDtypeStruct((NUM_ACTIONS, BATCH), jnp.float32),
        mesh=mesh,
        scratch_types=[
            pltpu.VMEM((BPW,), jnp.int32),
            pltpu.VMEM((NRING, NUM_ACTIONS, 128), jnp.float32),
            pltpu.VMEM((NUM_ACTIONS, BPW), jnp.float32),
            pltpu.SemaphoreType.DMA((NRING,)),
            pltpu.SemaphoreType.DMA,
        ],
        compiler_params=pltpu.CompilerParams(needs_layout_passes=False),
    )
    outT = run(obs, masksT, qT)
    return outT.T
